# Initial kernel scaffold; baseline (speedup 1.0000x reference)
import jax
import jax.numpy as jnp
from jax.experimental import pallas as pl


def kernel(states_avg, scores, dist_table, speaker_table, W1, b1, W2, b2, mention_ids, antecedent_ids, distance_buckets, speakers):
    raise NotImplementedError("write your pallas kernel here")


# speedup vs baseline: 4.1082x; 4.1082x over previous
import functools

import jax
import jax.numpy as jnp
from jax import lax
from jax.experimental import pallas as pl
from jax.experimental.pallas import tpu as pltpu
from jax.experimental.pallas import tpu_sc as plsc

NSP = 8192
NP = 65536
D = 512
NC = 2
NS = 16
NW = NC * NS
PPW = NP // NW
CHUNK = 32
NCHUNK = PPW // CHUNK
CH3 = 512
BLK2 = 512
F32 = jnp.float32
I32 = jnp.int32

_mesh = plsc.VectorSubcoreMesh(core_axis_name="c", subcore_axis_name="s",
                               num_cores=NC, num_subcores=NS)


def _precompute_body(s_ref, w1m_ref, w1a_ref, sa1_ref, sa2_ref):
    s = s_ref[...]
    sa1_ref[...] = jnp.dot(s, w1m_ref[...], preferred_element_type=F32)
    sa2_ref[...] = jnp.dot(s, w1a_ref[...], preferred_element_type=F32)


def _precompute(states, w1m, w1a):
    blk = 1024
    return pl.pallas_call(
        _precompute_body,
        grid=(NSP // blk,),
        in_specs=[
            pl.BlockSpec((blk, D), lambda i: (i, 0)),
            pl.BlockSpec((D, D), lambda i: (0, 0)),
            pl.BlockSpec((D, D), lambda i: (0, 0)),
        ],
        out_specs=[
            pl.BlockSpec((blk, D), lambda i: (i, 0)),
            pl.BlockSpec((blk, D), lambda i: (i, 0)),
        ],
        out_shape=[
            jax.ShapeDtypeStruct((NSP, D), F32),
            jax.ShapeDtypeStruct((NSP, D), F32),
        ],
        interpret=False,
    )(states, w1m, w1a)


def _phi_body(d_ref, s_ref, w1phi_ref, b1_ref, phi_ref):
    c = lax.broadcasted_iota(I32, (32, 1), 0)
    d_idx = c // 3
    s_idx = c - d_idx * 3
    oh_d = (d_idx == lax.broadcasted_iota(I32, (32, 10), 1)).astype(F32)
    oh_s = (s_idx == lax.broadcasted_iota(I32, (32, 3), 1)).astype(F32)
    emb = jnp.concatenate(
        [jnp.dot(oh_d, d_ref[...], preferred_element_type=F32),
         jnp.dot(oh_s, s_ref[...], preferred_element_type=F32)], axis=1)
    phi_ref[...] = (
        jnp.dot(emb, w1phi_ref[...], preferred_element_type=F32)
        + b1_ref[...][None, :])


def _phi_table(dist_table, speaker_table, w1phi, b1):
    return pl.pallas_call(
        _phi_body,
        out_shape=jax.ShapeDtypeStruct((32, D), F32),
        interpret=False,
    )(dist_table, speaker_table, w1phi, b1)


def _gather_body(states_hbm, sa1_hbm, sa2_hbm, phi_hbm, scores_hbm,
                 mid_hbm, aid_hbm, cmb_hbm,
                 prod_hbm, g_hbm, ssum_hbm,
                 mid_v, aid_v, cmb_v, m_v, a_v, s1_v, s2_v,
                 phi_v, scores_v, ssum_v, sem):
    wid = lax.axis_index("s") * NC + lax.axis_index("c")
    base = wid * PPW
    pltpu.sync_copy(phi_hbm, phi_v)
    pltpu.sync_copy(scores_hbm, scores_v)
    iota16 = lax.iota(I32, 16)

    def chunk(ci, carry):
        off = base + ci * CHUNK
        pltpu.sync_copy(mid_hbm.at[pl.ds(off, CHUNK)], mid_v)
        pltpu.sync_copy(aid_hbm.at[pl.ds(off, CHUNK)], aid_v)
        pltpu.sync_copy(cmb_hbm.at[pl.ds(off, CHUNK)], cmb_v)
        c1 = pltpu.async_copy(states_hbm.at[mid_v], m_v, sem)
        c2 = pltpu.async_copy(states_hbm.at[aid_v], a_v, sem)
        c3 = pltpu.async_copy(sa1_hbm.at[mid_v], s1_v, sem)
        c4 = pltpu.async_copy(sa2_hbm.at[aid_v], s2_v, sem)
        c1.wait()
        c2.wait()
        c3.wait()
        c4.wait()

        def pair(j, carry2):
            cj = plsc.load_gather(cmb_v, [jnp.zeros((16,), I32) + j])
            for k in range(D // 16):
                sl = pl.ds(k * 16, 16)
                m16 = m_v[j, sl]
                a16 = a_v[j, sl]
                m_v[j, sl] = m16 * a16
                phi16 = plsc.load_gather(phi_v, [cj + (k * 16) + iota16])
                s1_v[j, sl] = s1_v[j, sl] + s2_v[j, sl] + phi16
            return carry2

        lax.fori_loop(0, CHUNK, pair, 0)

        def grp(gi, carry2):
            sl = pl.ds(gi * 16, 16)
            sm = plsc.load_gather(scores_v, [mid_v[sl]])
            sa = plsc.load_gather(scores_v, [aid_v[sl]])
            ssum_v[sl] = sm + sa
            return carry2

        lax.fori_loop(0, CHUNK // 16, grp, 0)

        pltpu.sync_copy(m_v, prod_hbm.at[pl.ds(off, CHUNK)])
        pltpu.sync_copy(s1_v, g_hbm.at[pl.ds(off, CHUNK)])
        pltpu.sync_copy(ssum_v, ssum_hbm.at[pl.ds(off, CHUNK)])
        return carry

    lax.fori_loop(0, NCHUNK, chunk, 0)


def _gather(states, sa1, sa2, phi_flat, scores_flat, mid, aid, cmb):
    fn = pl.kernel(
        _gather_body,
        out_type=(
            jax.ShapeDtypeStruct((NP, D), F32),
            jax.ShapeDtypeStruct((NP, D), F32),
            jax.ShapeDtypeStruct((NP,), F32),
        ),
        mesh=_mesh,
        compiler_params=pltpu.CompilerParams(needs_layout_passes=False),
        scratch_types=[
            pltpu.VMEM((CHUNK,), I32),
            pltpu.VMEM((CHUNK,), I32),
            pltpu.VMEM((CHUNK,), I32),
            pltpu.VMEM((CHUNK, D), F32),
            pltpu.VMEM((CHUNK, D), F32),
            pltpu.VMEM((CHUNK, D), F32),
            pltpu.VMEM((CHUNK, D), F32),
            pltpu.VMEM((32 * D,), F32),
            pltpu.VMEM((NSP,), F32),
            pltpu.VMEM((CHUNK,), F32),
            pltpu.SemaphoreType.DMA,
        ],
    )
    return fn(states, sa1, sa2, phi_flat, scores_flat, mid, aid, cmb)


def _mlp_body(p_ref, g_ref, w1p_ref, w2_ref, ssum_ref, b2_ref,
              coref_ref, expl_ref):
    h = jnp.maximum(
        jnp.dot(p_ref[...], w1p_ref[...], preferred_element_type=F32)
        + g_ref[...], 0.0)
    ps = jnp.sum(h * w2_ref[...][None, :], axis=1)
    cs = ps + ssum_ref[0, 0, :] + b2_ref[0]
    coref_ref[0, 0, :] = cs
    expl_ref[0, 0, :] = jnp.exp(cs)


def _mlp(prod, g, w1p, w2_flat, ssum3, b2):
    nblk = NP // BLK2
    return pl.pallas_call(
        _mlp_body,
        grid=(nblk,),
        in_specs=[
            pl.BlockSpec((BLK2, D), lambda i: (i, 0)),
            pl.BlockSpec((BLK2, D), lambda i: (i, 0)),
            pl.BlockSpec((D, D), lambda i: (0, 0)),
            pl.BlockSpec((D,), lambda i: (0,)),
            pl.BlockSpec((1, 1, BLK2), lambda i: (i, 0, 0)),
            pl.BlockSpec(memory_space=pltpu.SMEM),
        ],
        out_specs=[
            pl.BlockSpec((1, 1, BLK2), lambda i: (i, 0, 0)),
            pl.BlockSpec((1, 1, BLK2), lambda i: (i, 0, 0)),
        ],
        out_shape=[
            jax.ShapeDtypeStruct((nblk, 1, BLK2), F32),
            jax.ShapeDtypeStruct((nblk, 1, BLK2), F32),
        ],
        interpret=False,
    )(prod, g, w1p, w2_flat, ssum3, b2)


def _segsum_body(expl_hbm, mid_hbm, part_hbm, acc_v, mid_v, val_v):
    wid = lax.axis_index("s") * NC + lax.axis_index("c")
    base = wid * PPW

    def zero(i, carry):
        acc_v[pl.ds(i * 16, 16)] = jnp.zeros((16,), F32)
        return carry

    lax.fori_loop(0, NSP // 16, zero, 0)

    def chunk(ci, carry):
        off = base + ci * CH3
        pltpu.sync_copy(mid_hbm.at[pl.ds(off, CH3)], mid_v)
        pltpu.sync_copy(expl_hbm.at[pl.ds(off, CH3)], val_v)

        def grp(gi, carry2):
            sl = pl.ds(gi * 16, 16)
            plsc.addupdate_scatter(acc_v, [mid_v[sl]], val_v[sl])
            return carry2

        lax.fori_loop(0, CH3 // 16, grp, 0)
        return carry

    lax.fori_loop(0, PPW // CH3, chunk, 0)
    pltpu.sync_copy(acc_v, part_hbm.at[wid])


def _segsum(expl, mid):
    fn = pl.kernel(
        _segsum_body,
        out_type=jax.ShapeDtypeStruct((NW, NSP), F32),
        mesh=_mesh,
        compiler_params=pltpu.CompilerParams(needs_layout_passes=False),
        scratch_types=[
            pltpu.VMEM((NSP,), F32),
            pltpu.VMEM((CH3,), I32),
            pltpu.VMEM((CH3,), F32),
        ],
    )
    return fn(expl, mid)


def _denom_body(part_ref, r_ref):
    r_ref[...] = 1.0 / (jnp.sum(part_ref[...], axis=0) + 1.0)


def _denom(part):
    return pl.pallas_call(
        _denom_body,
        out_shape=jax.ShapeDtypeStruct((NSP,), F32),
        interpret=False,
    )(part)


def _probs_body(expl_hbm, mid_hbm, r_hbm, out_hbm, r_v, mid_v, val_v):
    wid = lax.axis_index("s") * NC + lax.axis_index("c")
    base = wid * PPW
    pltpu.sync_copy(r_hbm, r_v)

    def chunk(ci, carry):
        off = base + ci * CH3
        pltpu.sync_copy(mid_hbm.at[pl.ds(off, CH3)], mid_v)
        pltpu.sync_copy(expl_hbm.at[pl.ds(off, CH3)], val_v)

        def grp(gi, carry2):
            sl = pl.ds(gi * 16, 16)
            rg = plsc.load_gather(r_v, [mid_v[sl]])
            val_v[sl] = val_v[sl] * rg
            return carry2

        lax.fori_loop(0, CH3 // 16, grp, 0)
        pltpu.sync_copy(val_v, out_hbm.at[pl.ds(off, CH3)])
        return carry

    lax.fori_loop(0, PPW // CH3, chunk, 0)


def _probs(expl, mid, r):
    fn = pl.kernel(
        _probs_body,
        out_type=jax.ShapeDtypeStruct((NP,), F32),
        mesh=_mesh,
        compiler_params=pltpu.CompilerParams(needs_layout_passes=False),
        scratch_types=[
            pltpu.VMEM((NSP,), F32),
            pltpu.VMEM((CH3,), I32),
            pltpu.VMEM((CH3,), F32),
        ],
    )
    return fn(expl, mid, r)


def kernel(states_avg, scores, dist_table, speaker_table, W1, b1, W2, b2,
           mention_ids, antecedent_ids, distance_buckets, speakers):
    w1m = W1[0:D]
    w1a = W1[D:2 * D]
    w1p = W1[2 * D:3 * D]
    w1phi = W1[3 * D:]
    mid = mention_ids.astype(I32)
    aid = antecedent_ids.astype(I32)
    cmb = (distance_buckets.astype(I32) * 3 + speakers.astype(I32)) * D

    sa1, sa2 = _precompute(states_avg, w1m, w1a)
    phi = _phi_table(dist_table, speaker_table, w1phi, b1)
    prod, g, ssum = _gather(states_avg, sa1, sa2, phi.reshape(-1),
                            scores[:, 0], mid, aid, cmb)
    coref3, expl3 = _mlp(prod, g, w1p, W2[:, 0],
                         ssum.reshape(NP // BLK2, 1, BLK2), b2)
    expl = expl3.reshape(NP)
    part = _segsum(expl, mid)
    r = _denom(part)
    probs = _probs(expl, mid, r)
    return coref3.reshape(NP, 1), probs, r


# speedup vs baseline: 6.5641x; 1.5978x over previous
import functools

import jax
import jax.numpy as jnp
from jax import lax
from jax.experimental import pallas as pl
from jax.experimental.pallas import tpu as pltpu
from jax.experimental.pallas import tpu_sc as plsc

NSP = 8192
NP = 65536
D = 512
NC = 2
NS = 16
NW = NC * NS
PPW = NP // NW
CHUNK = 16
NCH2 = PPW // (2 * CHUNK)
CH3 = 512
BLK2 = 512
F32 = jnp.float32
I32 = jnp.int32

_mesh = plsc.VectorSubcoreMesh(core_axis_name="c", subcore_axis_name="s",
                               num_cores=NC, num_subcores=NS)


def _precompute_body(s_ref, w1m_ref, w1a_ref, sa1_ref, sa2_ref):
    s = s_ref[...]
    sa1_ref[...] = jnp.dot(s, w1m_ref[...], preferred_element_type=F32)
    sa2_ref[...] = jnp.dot(s, w1a_ref[...], preferred_element_type=F32)


def _precompute(states, w1m, w1a):
    blk = 1024
    return pl.pallas_call(
        _precompute_body,
        grid=(NSP // blk,),
        in_specs=[
            pl.BlockSpec((blk, D), lambda i: (i, 0)),
            pl.BlockSpec((D, D), lambda i: (0, 0)),
            pl.BlockSpec((D, D), lambda i: (0, 0)),
        ],
        out_specs=[
            pl.BlockSpec((blk, D), lambda i: (i, 0)),
            pl.BlockSpec((blk, D), lambda i: (i, 0)),
        ],
        out_shape=[
            jax.ShapeDtypeStruct((NSP, D), F32),
            jax.ShapeDtypeStruct((NSP, D), F32),
        ],
        interpret=False,
    )(states, w1m, w1a)


def _phi_body(d_ref, s_ref, w1phi_ref, b1_ref, phi_ref):
    c = lax.broadcasted_iota(I32, (32, 1), 0)
    d_idx = c // 3
    s_idx = c - d_idx * 3
    oh_d = (d_idx == lax.broadcasted_iota(I32, (32, 10), 1)).astype(F32)
    oh_s = (s_idx == lax.broadcasted_iota(I32, (32, 3), 1)).astype(F32)
    emb = jnp.concatenate(
        [jnp.dot(oh_d, d_ref[...], preferred_element_type=F32),
         jnp.dot(oh_s, s_ref[...], preferred_element_type=F32)], axis=1)
    phi_ref[...] = (
        jnp.dot(emb, w1phi_ref[...], preferred_element_type=F32)
        + b1_ref[...][None, :])


def _phi_table(dist_table, speaker_table, w1phi, b1):
    return pl.pallas_call(
        _phi_body,
        out_shape=jax.ShapeDtypeStruct((32, D), F32),
        interpret=False,
    )(dist_table, speaker_table, w1phi, b1)


def _gather_body(states_hbm, sa1_hbm, sa2_hbm, scores_hbm, mid_hbm, aid_hbm,
                 prod_hbm, g_hbm, ssum_hbm,
                 mid_v, aid_v, scores_v, ssum_v,
                 m0, a0, p0, q0, m1, a1, p1, q1,
                 gsem0, gsem1, osem):
    wid = lax.axis_index("s") * NC + lax.axis_index("c")
    base = wid * PPW
    pltpu.sync_copy(scores_hbm, scores_v)
    pltpu.sync_copy(mid_hbm.at[pl.ds(base, PPW)], mid_v)
    pltpu.sync_copy(aid_hbm.at[pl.ds(base, PPW)], aid_v)

    def sgrp(gi, carry):
        sl = pl.ds(gi * 16, 16)
        ssum_v[sl] = (plsc.load_gather(scores_v, [mid_v[sl]])
                      + plsc.load_gather(scores_v, [aid_v[sl]]))
        return carry

    lax.fori_loop(0, PPW // 16, sgrp, 0)
    pltpu.sync_copy(ssum_v, ssum_hbm.at[pl.ds(base, PPW)])

    def issue(ci, bufs, sem):
        lsl = pl.ds(ci * CHUNK, CHUNK)
        mid16 = mid_v[lsl]
        aid16 = aid_v[lsl]
        return [pltpu.async_copy(states_hbm.at[mid16], bufs[0], sem),
                pltpu.async_copy(states_hbm.at[aid16], bufs[1], sem),
                pltpu.async_copy(sa1_hbm.at[mid16], bufs[2], sem),
                pltpu.async_copy(sa2_hbm.at[aid16], bufs[3], sem)]

    def drain(sem, n):
        for _ in range(n):
            pltpu.make_async_copy(states_hbm.at[pl.ds(0, CHUNK)], m0,
                                  sem).wait()

    def vpass(mb, ab, pb, qb):
        def pair(j, carry):
            for k in range(D // 16):
                sl = pl.ds(k * 16, 16)
                mb[j, sl] = mb[j, sl] * ab[j, sl]
                pb[j, sl] = pb[j, sl] + qb[j, sl]
            return carry

        lax.fori_loop(0, CHUNK, pair, 0)

    set0 = (m0, a0, p0, q0)
    set1 = (m1, a1, p1, q1)
    issue(0, set0, gsem0)

    def dchunk(t, carry):
        c0 = 2 * t
        off0 = base + c0 * CHUNK
        off1 = off0 + CHUNK
        drain(gsem0, 4)

        @pl.when(t > 0)
        def _():
            drain(osem, 2)

        d1 = issue(c0 + 1, set1, gsem1)
        vpass(*set0)
        o1 = pltpu.async_copy(m0, prod_hbm.at[pl.ds(off0, CHUNK)], osem)
        o2 = pltpu.async_copy(p0, g_hbm.at[pl.ds(off0, CHUNK)], osem)
        o1.wait()
        o2.wait()

        @pl.when(t < NCH2 - 1)
        def _():
            issue(c0 + 2, set0, gsem0)

        for d in d1:
            d.wait()
        vpass(*set1)
        pltpu.async_copy(m1, prod_hbm.at[pl.ds(off1, CHUNK)], osem)
        pltpu.async_copy(p1, g_hbm.at[pl.ds(off1, CHUNK)], osem)
        return carry

    lax.fori_loop(0, NCH2, dchunk, 0)
    drain(osem, 2)


def _gather(states, sa1, sa2, scores_flat, mid, aid):
    buf = lambda: pltpu.VMEM((CHUNK, D), F32)
    fn = pl.kernel(
        _gather_body,
        out_type=(
            jax.ShapeDtypeStruct((NP, D), F32),
            jax.ShapeDtypeStruct((NP, D), F32),
            jax.ShapeDtypeStruct((NP,), F32),
        ),
        mesh=_mesh,
        compiler_params=pltpu.CompilerParams(needs_layout_passes=False),
        scratch_types=[
            pltpu.VMEM((PPW,), I32),
            pltpu.VMEM((PPW,), I32),
            pltpu.VMEM((NSP,), F32),
            pltpu.VMEM((PPW,), F32),
            buf(), buf(), buf(), buf(),
            buf(), buf(), buf(), buf(),
            pltpu.SemaphoreType.DMA,
            pltpu.SemaphoreType.DMA,
            pltpu.SemaphoreType.DMA,
        ],
    )
    return fn(states, sa1, sa2, scores_flat, mid, aid)


def _mlp_body(p_ref, g_ref, w1p_ref, phi_ref, w2_ref, ssum_ref, cmb_ref,
              b2_ref, coref_ref, expl_ref):
    ohT = (lax.broadcasted_iota(I32, (32, BLK2), 0)
           == cmb_ref[0, :, :]).astype(F32)
    pt = lax.dot_general(ohT, phi_ref[...],
                         dimension_numbers=(((0,), (0,)), ((), ())),
                         preferred_element_type=F32)
    h = jnp.maximum(
        jnp.dot(p_ref[...], w1p_ref[...], preferred_element_type=F32)
        + g_ref[...] + pt, 0.0)
    ps = jnp.sum(h * w2_ref[...][None, :], axis=1)
    cs = ps + ssum_ref[0, 0, :] + b2_ref[0]
    coref_ref[0, 0, :] = cs
    expl_ref[0, 0, :] = jnp.exp(cs)


def _mlp(prod, g, w1p, phi, w2_flat, ssum3, cmb3, b2):
    nblk = NP // BLK2
    return pl.pallas_call(
        _mlp_body,
        grid=(nblk,),
        in_specs=[
            pl.BlockSpec((BLK2, D), lambda i: (i, 0)),
            pl.BlockSpec((BLK2, D), lambda i: (i, 0)),
            pl.BlockSpec((D, D), lambda i: (0, 0)),
            pl.BlockSpec((32, D), lambda i: (0, 0)),
            pl.BlockSpec((D,), lambda i: (0,)),
            pl.BlockSpec((1, 1, BLK2), lambda i: (i, 0, 0)),
            pl.BlockSpec((1, 1, BLK2), lambda i: (i, 0, 0)),
            pl.BlockSpec(memory_space=pltpu.SMEM),
        ],
        out_specs=[
            pl.BlockSpec((1, 1, BLK2), lambda i: (i, 0, 0)),
            pl.BlockSpec((1, 1, BLK2), lambda i: (i, 0, 0)),
        ],
        out_shape=[
            jax.ShapeDtypeStruct((nblk, 1, BLK2), F32),
            jax.ShapeDtypeStruct((nblk, 1, BLK2), F32),
        ],
        interpret=False,
    )(prod, g, w1p, phi, w2_flat, ssum3, cmb3, b2)


def _segsum_body(expl_hbm, mid_hbm, part_hbm, acc_v, mid_v, val_v):
    wid = lax.axis_index("s") * NC + lax.axis_index("c")
    base = wid * PPW

    def zero(i, carry):
        acc_v[pl.ds(i * 16, 16)] = jnp.zeros((16,), F32)
        return carry

    lax.fori_loop(0, NSP // 16, zero, 0)

    def chunk(ci, carry):
        off = base + ci * CH3
        pltpu.sync_copy(mid_hbm.at[pl.ds(off, CH3)], mid_v)
        pltpu.sync_copy(expl_hbm.at[pl.ds(off, CH3)], val_v)

        def grp(gi, carry2):
            sl = pl.ds(gi * 16, 16)
            plsc.addupdate_scatter(acc_v, [mid_v[sl]], val_v[sl])
            return carry2

        lax.fori_loop(0, CH3 // 16, grp, 0)
        return carry

    lax.fori_loop(0, PPW // CH3, chunk, 0)
    pltpu.sync_copy(acc_v, part_hbm.at[wid])


def _segsum(expl, mid):
    fn = pl.kernel(
        _segsum_body,
        out_type=jax.ShapeDtypeStruct((NW, NSP), F32),
        mesh=_mesh,
        compiler_params=pltpu.CompilerParams(needs_layout_passes=False),
        scratch_types=[
            pltpu.VMEM((NSP,), F32),
            pltpu.VMEM((CH3,), I32),
            pltpu.VMEM((CH3,), F32),
        ],
    )
    return fn(expl, mid)


def _denom_body(part_ref, r_ref):
    r_ref[...] = 1.0 / (jnp.sum(part_ref[...], axis=0) + 1.0)


def _denom(part):
    return pl.pallas_call(
        _denom_body,
        out_shape=jax.ShapeDtypeStruct((NSP,), F32),
        interpret=False,
    )(part)


def _probs_body(expl_hbm, mid_hbm, r_hbm, out_hbm, r_v, mid_v, val_v):
    wid = lax.axis_index("s") * NC + lax.axis_index("c")
    base = wid * PPW
    pltpu.sync_copy(r_hbm, r_v)

    def chunk(ci, carry):
        off = base + ci * CH3
        pltpu.sync_copy(mid_hbm.at[pl.ds(off, CH3)], mid_v)
        pltpu.sync_copy(expl_hbm.at[pl.ds(off, CH3)], val_v)

        def grp(gi, carry2):
            sl = pl.ds(gi * 16, 16)
            rg = plsc.load_gather(r_v, [mid_v[sl]])
            val_v[sl] = val_v[sl] * rg
            return carry2

        lax.fori_loop(0, CH3 // 16, grp, 0)
        pltpu.sync_copy(val_v, out_hbm.at[pl.ds(off, CH3)])
        return carry

    lax.fori_loop(0, PPW // CH3, chunk, 0)


def _probs(expl, mid, r):
    fn = pl.kernel(
        _probs_body,
        out_type=jax.ShapeDtypeStruct((NP,), F32),
        mesh=_mesh,
        compiler_params=pltpu.CompilerParams(needs_layout_passes=False),
        scratch_types=[
            pltpu.VMEM((NSP,), F32),
            pltpu.VMEM((CH3,), I32),
            pltpu.VMEM((CH3,), F32),
        ],
    )
    return fn(expl, mid, r)


def kernel(states_avg, scores, dist_table, speaker_table, W1, b1, W2, b2,
           mention_ids, antecedent_ids, distance_buckets, speakers):
    w1m = W1[0:D]
    w1a = W1[D:2 * D]
    w1p = W1[2 * D:3 * D]
    w1phi = W1[3 * D:]
    mid = mention_ids.astype(I32)
    aid = antecedent_ids.astype(I32)
    cmb = distance_buckets.astype(I32) * 3 + speakers.astype(I32)

    sa1, sa2 = _precompute(states_avg, w1m, w1a)
    phi = _phi_table(dist_table, speaker_table, w1phi, b1)
    prod, g, ssum = _gather(states_avg, sa1, sa2, scores[:, 0], mid, aid)
    coref3, expl3 = _mlp(prod, g, w1p, phi, W2[:, 0],
                         ssum.reshape(NP // BLK2, 1, BLK2),
                         cmb.reshape(NP // BLK2, 1, BLK2), b2)
    expl = expl3.reshape(NP)
    part = _segsum(expl, mid)
    r = _denom(part)
    probs = _probs(expl, mid, r)
    return coref3.reshape(NP, 1), probs, r
